# baseline (speedup 1.0000x reference)
import functools
import math

import jax
import jax.numpy as jnp
from jax import lax
from jax.experimental import pallas as pl
from jax.experimental.pallas import tpu as pltpu
from jax.experimental.pallas import tpu_sc as plsc

D_MODEL = 64
SCALE = math.sqrt(D_MODEL)


def _scale_transpose_table(lut_t, V):
    C = 8192

    def body(in_ref, out_ref):
        eye = SCALE * jnp.eye(64, dtype=jnp.float32)
        z = lax.dot_general(in_ref[...], eye, (((0,), (0,)), ((), ())),
                            preferred_element_type=jnp.float32)
        w = z.reshape(C // 2, 2, 64)
        out_ref[:, 0:64] = w[:, 0, :]
        out_ref[:, 64:128] = w[:, 1, :]

    return pl.pallas_call(
        body,
        grid=(pl.cdiv(V, C),),
        in_specs=[pl.BlockSpec((64, C), lambda i: (0, i))],
        out_specs=pl.BlockSpec((C // 2, 128), lambda i: (i, 0)),
        out_shape=jax.ShapeDtypeStruct((V // 2, 128), jnp.float32),
        compiler_params=pltpu.CompilerParams(
            fuse_transposed_lhs_in_matmul=True),
    )(lut_t)


@functools.lru_cache(maxsize=None)
def _make_gather(B, V, D, T_DIM):
    S = B // T_DIM
    NTC = S // 128
    n_groups = T_DIM * NTC
    info = plsc.get_sparse_core_info()
    NW = info.num_cores * info.num_subcores
    g_per_w = n_groups // NW
    mesh = plsc.VectorSubcoreMesh(core_axis_name="c", subcore_axis_name="s")

    @functools.partial(
        pl.kernel,
        mesh=mesh,
        compiler_params=pltpu.CompilerParams(
            use_tc_tiling_on_sc=False, needs_layout_passes=False),
        out_type=jax.ShapeDtypeStruct((T_DIM * 8 * NTC * 1024,), jnp.float32),
        scratch_types=[
            pltpu.VMEM((8 * 128,), jnp.int32),
            pltpu.VMEM((128, D), jnp.float32),
            pltpu.VMEM((128, D), jnp.float32),
        ] + [pltpu.VMEM((8 * 1024,), jnp.float32) for _ in range(8)] + [
            pltpu.SemaphoreType.DMA,
            pltpu.SemaphoreType.DMA,
        ],
    )
    def k(idx_hbm, tab_hbm, out_hbm, idx_v, ga_v, gb_v,
          t0, t1, t2, t3, t4, t5, t6, t7, sem_g, sem_o):
        wid = lax.axis_index("s") * info.num_cores + lax.axis_index("c")
        g0 = wid * g_per_w
        gbufs = [ga_v, gb_v]
        tbufs = [t0, t1, t2, t3, t4, t5, t6, t7]
        lane = lax.iota(jnp.int32, 16)
        base = [((16 * j + lane) // 8) * 1024 + ((16 * j + lane) % 8) * 128
                for j in range(D // 16)]
        n_sb = g_per_w // 8

        def sb_body(sb, carry):
            gid0 = g0 + sb * 8
            pltpu.sync_copy(idx_hbm.at[pl.ds(gid0 * 128, 8 * 128)], idx_v)
            waits = []
            gh = pltpu.async_copy(
                tab_hbm.at[idx_v.at[pl.ds(0, 128)]], gbufs[0], sem_g)
            for u in range(8):
                if u < 7:
                    gh_next = pltpu.async_copy(
                        tab_hbm.at[idx_v.at[pl.ds((u + 1) * 128, 128)]],
                        gbufs[(u + 1) & 1], sem_g)
                gh.wait()
                g_v = gbufs[u & 1]
                t_v = tbufs[u]

                @functools.partial(plsc.parallel_loop, 0, 128, unroll=4)
                def s_body(s):
                    vals = [g_v[s, pl.ds(16 * j, 16)] for j in range(D // 16)]
                    for j in range(D // 16):
                        plsc.store_scatter(t_v, [base[j] + s], vals[j])
                gid = gid0 + u
                out_base = (gid // NTC) * (8 * NTC * 1024) + (gid % NTC) * 1024
                for k_i in range(8):
                    waits.append(pltpu.async_copy(
                        t_v.at[pl.ds(k_i * 1024, 1024)],
                        out_hbm.at[pl.ds(out_base + k_i * NTC * 1024, 1024)],
                        sem_o))
                if u < 7:
                    gh = gh_next
            for h in waits:
                h.wait()
            return carry

        lax.fori_loop(0, n_sb, sb_body, 0)

    return k


def kernel(x, lut):
    S, T_DIM = x.shape
    V, D = lut.shape
    idx = x.T.reshape(-1).astype(jnp.int32)
    tab = _scale_transpose_table(lut.T, V)
    tab = tab.reshape(V, D)
    out = _make_gather(S * T_DIM, V, D, T_DIM)(idx, tab)
    out = out.reshape(T_DIM, 8, S // 128, 8, 128)
    return out.transpose(2, 4, 0, 1, 3).reshape(S, T_DIM, D)


# speedup vs baseline: 1.0161x; 1.0161x over previous
import functools
import math

import jax
import jax.numpy as jnp
from jax import lax
from jax.experimental import pallas as pl
from jax.experimental.pallas import tpu as pltpu
from jax.experimental.pallas import tpu_sc as plsc

D_MODEL = 64
SCALE = math.sqrt(D_MODEL)


def _scale_transpose_table(lut_t, V):
    C = 16384

    def body(in_ref, out_ref):
        eye = SCALE * jnp.eye(64, dtype=jnp.float32)
        z = lax.dot_general(in_ref[...], eye, (((0,), (0,)), ((), ())),
                            preferred_element_type=jnp.float32)
        w = z.reshape(C // 2, 2, 64)
        out_ref[:, 0:64] = w[:, 0, :]
        out_ref[:, 64:128] = w[:, 1, :]

    return pl.pallas_call(
        body,
        grid=(pl.cdiv(V, C),),
        in_specs=[pl.BlockSpec((64, C), lambda i: (0, i))],
        out_specs=pl.BlockSpec((C // 2, 128), lambda i: (i, 0)),
        out_shape=jax.ShapeDtypeStruct((V // 2, 128), jnp.float32),
    )(lut_t)


@functools.lru_cache(maxsize=None)
def _make_gather(B, V, D, T_DIM):
    S = B // T_DIM
    NTC = S // 128
    n_groups = T_DIM * NTC
    info = plsc.get_sparse_core_info()
    NW = info.num_cores * info.num_subcores
    g_per_w = n_groups // NW
    mesh = plsc.VectorSubcoreMesh(core_axis_name="c", subcore_axis_name="s")

    @functools.partial(
        pl.kernel,
        mesh=mesh,
        compiler_params=pltpu.CompilerParams(
            use_tc_tiling_on_sc=False, needs_layout_passes=False),
        out_type=jax.ShapeDtypeStruct((T_DIM * 8 * NTC * 1024,), jnp.float32),
        scratch_types=[
            pltpu.VMEM((8 * 128,), jnp.int32),
            pltpu.VMEM((128, D), jnp.float32),
            pltpu.VMEM((128, D), jnp.float32),
        ] + [pltpu.VMEM((8 * 1024,), jnp.float32) for _ in range(8)] + [
            pltpu.SemaphoreType.DMA,
            pltpu.SemaphoreType.DMA,
        ],
    )
    def k(idx_hbm, tab_hbm, out_hbm, idx_v, ga_v, gb_v,
          t0, t1, t2, t3, t4, t5, t6, t7, sem_g, sem_o):
        wid = lax.axis_index("s") * info.num_cores + lax.axis_index("c")
        g0 = wid * g_per_w
        gbufs = [ga_v, gb_v]
        tbufs = [t0, t1, t2, t3, t4, t5, t6, t7]
        lane = lax.iota(jnp.int32, 16)
        base = [((16 * j + lane) // 8) * 1024 + ((16 * j + lane) % 8) * 128
                for j in range(D // 16)]
        n_sb = g_per_w // 8

        def sb_body(sb, carry):
            gid0 = g0 + sb * 8
            pltpu.sync_copy(idx_hbm.at[pl.ds(gid0 * 128, 8 * 128)], idx_v)
            waits = []
            gh = pltpu.async_copy(
                tab_hbm.at[idx_v.at[pl.ds(0, 128)]], gbufs[0], sem_g)
            for u in range(8):
                if u < 7:
                    gh_next = pltpu.async_copy(
                        tab_hbm.at[idx_v.at[pl.ds((u + 1) * 128, 128)]],
                        gbufs[(u + 1) & 1], sem_g)
                gh.wait()
                g_v = gbufs[u & 1]
                t_v = tbufs[u]

                @functools.partial(plsc.parallel_loop, 0, 128, unroll=4)
                def s_body(s):
                    vals = [g_v[s, pl.ds(16 * j, 16)] for j in range(D // 16)]
                    for j in range(D // 16):
                        plsc.store_scatter(t_v, [base[j] + s], vals[j])
                gid = gid0 + u
                out_base = (gid // NTC) * (8 * NTC * 1024) + (gid % NTC) * 1024
                for k_i in range(8):
                    waits.append(pltpu.async_copy(
                        t_v.at[pl.ds(k_i * 1024, 1024)],
                        out_hbm.at[pl.ds(out_base + k_i * NTC * 1024, 1024)],
                        sem_o))
                if u < 7:
                    gh = gh_next
            for h in waits:
                h.wait()
            return carry

        lax.fori_loop(0, n_sb, sb_body, 0)

    return k


def kernel(x, lut):
    S, T_DIM = x.shape
    V, D = lut.shape
    idx = x.T.reshape(-1).astype(jnp.int32)
    tab = _scale_transpose_table(lut.T, V)
    tab = tab.reshape(V, D)
    out = _make_gather(S * T_DIM, V, D, T_DIM)(idx, tab)
    out = out.reshape(T_DIM, 8, S // 128, 8, 128)
    return out.transpose(2, 4, 0, 1, 3).reshape(S, T_DIM, D)


# speedup vs baseline: 1.0689x; 1.0520x over previous
import functools
import math

import jax
import jax.numpy as jnp
from jax import lax
from jax.experimental import pallas as pl
from jax.experimental.pallas import tpu as pltpu
from jax.experimental.pallas import tpu_sc as plsc

D_MODEL = 64
SCALE = math.sqrt(D_MODEL)


def _scale_transpose_table(lut_t, V):
    C = 16384

    def body(in_ref, out_ref):
        z = in_ref[...].T * SCALE
        w = z.reshape(C // 2, 2, 64)
        out_ref[:, 0:64] = w[:, 0, :]
        out_ref[:, 64:128] = w[:, 1, :]

    return pl.pallas_call(
        body,
        grid=(pl.cdiv(V, C),),
        in_specs=[pl.BlockSpec((64, C), lambda i: (0, i))],
        out_specs=pl.BlockSpec((C // 2, 128), lambda i: (i, 0)),
        out_shape=jax.ShapeDtypeStruct((V // 2, 128), jnp.float32),
    )(lut_t)


@functools.lru_cache(maxsize=None)
def _make_gather(B, V, D, T_DIM):
    S = B // T_DIM
    NTC = S // 128
    n_groups = T_DIM * NTC
    info = plsc.get_sparse_core_info()
    NW = info.num_cores * info.num_subcores
    g_per_w = n_groups // NW
    mesh = plsc.VectorSubcoreMesh(core_axis_name="c", subcore_axis_name="s")

    @functools.partial(
        pl.kernel,
        mesh=mesh,
        compiler_params=pltpu.CompilerParams(
            use_tc_tiling_on_sc=False, needs_layout_passes=False),
        out_type=jax.ShapeDtypeStruct((T_DIM * 8 * NTC * 1024,), jnp.float32),
        scratch_types=[
            pltpu.VMEM((8 * 128,), jnp.int32),
            pltpu.VMEM((128, D), jnp.float32),
            pltpu.VMEM((128, D), jnp.float32),
        ] + [pltpu.VMEM((8 * 1024,), jnp.float32) for _ in range(8)] + [
            pltpu.SemaphoreType.DMA,
            pltpu.SemaphoreType.DMA,
        ],
    )
    def k(idx_hbm, tab_hbm, out_hbm, idx_v, ga_v, gb_v,
          t0, t1, t2, t3, t4, t5, t6, t7, sem_g, sem_o):
        wid = lax.axis_index("s") * info.num_cores + lax.axis_index("c")
        g0 = wid * g_per_w
        gbufs = [ga_v, gb_v]
        tbufs = [t0, t1, t2, t3, t4, t5, t6, t7]
        lane = lax.iota(jnp.int32, 16)
        base = [((16 * j + lane) // 8) * 1024 + ((16 * j + lane) % 8) * 128
                for j in range(D // 16)]
        n_sb = g_per_w // 8

        def sb_body(sb, carry):
            gid0 = g0 + sb * 8
            pltpu.sync_copy(idx_hbm.at[pl.ds(gid0 * 128, 8 * 128)], idx_v)
            waits = []
            gh = pltpu.async_copy(
                tab_hbm.at[idx_v.at[pl.ds(0, 128)]], gbufs[0], sem_g)
            for u in range(8):
                if u < 7:
                    gh_next = pltpu.async_copy(
                        tab_hbm.at[idx_v.at[pl.ds((u + 1) * 128, 128)]],
                        gbufs[(u + 1) & 1], sem_g)
                gh.wait()
                g_v = gbufs[u & 1]
                t_v = tbufs[u]

                @functools.partial(plsc.parallel_loop, 0, 128, unroll=4)
                def s_body(s):
                    vals = [g_v[s, pl.ds(16 * j, 16)] for j in range(D // 16)]
                    for j in range(D // 16):
                        plsc.store_scatter(t_v, [base[j] + s], vals[j])
                gid = gid0 + u
                out_base = (gid // NTC) * (8 * NTC * 1024) + (gid % NTC) * 1024
                for k_i in range(8):
                    waits.append(pltpu.async_copy(
                        t_v.at[pl.ds(k_i * 1024, 1024)],
                        out_hbm.at[pl.ds(out_base + k_i * NTC * 1024, 1024)],
                        sem_o))
                if u < 7:
                    gh = gh_next
            for h in waits:
                h.wait()
            return carry

        lax.fori_loop(0, n_sb, sb_body, 0)

    return k


def kernel(x, lut):
    S, T_DIM = x.shape
    V, D = lut.shape
    idx = x.T.reshape(-1).astype(jnp.int32)
    tab = _scale_transpose_table(lut.T, V)
    tab = tab.reshape(V, D)
    out = _make_gather(S * T_DIM, V, D, T_DIM)(idx, tab)
    out = out.reshape(T_DIM, 8, S // 128, 8, 128)
    return out.transpose(2, 4, 0, 1, 3).reshape(S, T_DIM, D)
